# baseline (speedup 1.0000x reference)
import functools

import jax
import jax.numpy as jnp
from jax import lax
from jax.experimental import pallas as pl
from jax.experimental.pallas import tpu as pltpu
from jax.experimental.pallas import tpu_sc as plsc

N_USERS = 20000
N_ITEMS = 30000
N_NODES = N_USERS + N_ITEMS
E = 800000
D = 64
DELTA = 0.8
N_LAYERS = 2

FCH = 8
FPC = FCH // 2
ER = 6400
EPAD = ER * 128
RPT = ER // 16
NBR = 8
NBATCH = RPT // NBR
N_PAD = 50048
NPT = N_PAD // 16


def _mm_body(x_ref, w_ref, b_ref, o_ref, *, normalize):
    y = jnp.dot(x_ref[...], w_ref[...], preferred_element_type=jnp.float32)
    y = y + b_ref[...]
    if normalize:
        n2 = jnp.sum(y * y, axis=1, keepdims=True)
        y = y * jax.lax.rsqrt(jnp.maximum(n2, 1e-24))
    o_ref[...] = y


def _mm(x, w, b, block_rows, normalize=False):
    m, k = x.shape
    n = w.shape[1]
    assert m % block_rows == 0
    return pl.pallas_call(
        functools.partial(_mm_body, normalize=normalize),
        grid=(m // block_rows,),
        in_specs=[
            pl.BlockSpec((block_rows, k), lambda i: (i, 0)),
            pl.BlockSpec((k, n), lambda i: (0, 0)),
            pl.BlockSpec((1, n), lambda i: (0, 0)),
        ],
        out_specs=pl.BlockSpec((block_rows, n), lambda i: (i, 0)),
        out_shape=jax.ShapeDtypeStruct((m, n), jnp.float32),
    )(x, w, b)


def _bcast_lane(vec, l):
    idx = jnp.full((16, 1), l, dtype=jnp.int32)
    dnums = lax.GatherDimensionNumbers(
        offset_dims=(), collapsed_slice_dims=(0,), start_index_map=(0,))
    return lax.gather(vec, idx, dnums, (1,),
                      mode=lax.GatherScatterMode.PROMISE_IN_BOUNDS)


def _prop_body(x2, srcp, dstr, wr, y2, sidx, didx, wv, rows, ibuf, acc,
               gsem, ssem, isem):
    c = lax.axis_index("c")
    s = lax.axis_index("s")
    row_lo = s * RPT
    node_lo = s * NPT
    EB = NBR * 128

    def fchunk_body(r, _):
        f = c * FPC + r
        fbase = f * N_PAD

        off = 0
        for sz in (512, 512, 512, 512, 512, 512, 56):
            pltpu.sync_copy(x2.at[pl.ds(fbase + node_lo + off, sz)],
                            ibuf.at[pl.ds(0, sz)])

            def init_body(i, _):
                ibuf[i] = ibuf[i] * DELTA
                return 0
            lax.fori_loop(0, sz, init_body, 0, unroll=4)
            pltpu.sync_copy(ibuf.at[pl.ds(0, sz)],
                            acc.at[pl.ds(node_lo + off, sz)])
            off += sz
        plsc.subcore_barrier()

        def fire_idx(t):
            buf = t % 4
            eb = (row_lo + t * NBR) * 128
            pltpu.async_copy(srcp.at[pl.ds(f * EPAD + eb, EB)], sidx.at[buf],
                             isem)
            pltpu.async_copy(dstr.at[pl.ds(eb, EB)], didx.at[buf], isem)
            pltpu.async_copy(wr.at[pl.ds(eb, EB)], wv.at[buf], isem)

        def drain_idx(t):
            buf = t % 4
            eb = (row_lo + t * NBR) * 128
            pltpu.make_async_copy(srcp.at[pl.ds(f * EPAD + eb, EB)],
                                  sidx.at[buf], isem).wait()
            pltpu.make_async_copy(dstr.at[pl.ds(eb, EB)], didx.at[buf],
                                  isem).wait()
            pltpu.make_async_copy(wr.at[pl.ds(eb, EB)], wv.at[buf],
                                  isem).wait()

        def fire_gathers(t):
            ib, rb = t % 4, t % 3
            for q in range(NBR):
                pltpu.async_copy(x2.at[sidx.at[ib, pl.ds(q * 128, 128)]],
                                 rows.at[rb, pl.ds(q * 128, 128)], gsem)

        def drain_gathers(t):
            ib, rb = t % 4, t % 3
            for q in range(NBR):
                pltpu.make_async_copy(x2.at[sidx.at[ib, pl.ds(q * 128, 128)]],
                                      rows.at[rb, pl.ds(q * 128, 128)],
                                      gsem).wait()

        def fire_scatter(t):
            ib, rb = t % 4, t % 3
            for q in range(NBR):
                pltpu.async_copy(rows.at[rb, pl.ds(q * 128, 128)],
                                 acc.at[didx.at[ib, pl.ds(q * 128, 128)]],
                                 ssem, add=True)

        def drain_scatter(t):
            ib, rb = t % 4, t % 3
            for q in range(NBR):
                pltpu.make_async_copy(rows.at[rb, pl.ds(q * 128, 128)],
                                      acc.at[didx.at[ib, pl.ds(q * 128, 128)]],
                                      ssem).wait()

        def compute(t):
            ib, rb = t % 4, t % 3

            def q_body(q, _):
                base = q * 128
                for k in range(8):
                    wk = wv[ib, pl.ds(base + k * 16, 16)]
                    for l in range(16):
                        i = base + k * 16 + l
                        rows[rb, i] = rows[rb, i] * _bcast_lane(wk, l)
                return 0
            lax.fori_loop(0, NBR, q_body, 0)

        fire_idx(0)
        drain_idx(0)
        fire_gathers(0)
        fire_idx(1)

        def batch_body(b, _):
            @pl.when(b + 2 < NBATCH)
            def _():
                fire_idx(b + 2)

            @pl.when(b + 1 < NBATCH)
            def _():
                drain_idx(b + 1)
                fire_gathers(b + 1)
            drain_gathers(b)
            compute(b)
            return 0
        lax.fori_loop(0, NBATCH, batch_body, 0)
        plsc.subcore_barrier()

        pltpu.sync_copy(acc.at[pl.ds(node_lo, NPT)],
                        y2.at[pl.ds(fbase + node_lo, NPT)])
        plsc.subcore_barrier()
        return 0

    lax.fori_loop(0, FPC, fchunk_body, 0)


def _propagate_layer(x2, srcp, dstr, wr):
    return pl.kernel(
        _prop_body,
        out_type=jax.ShapeDtypeStruct((FCH * N_PAD, 16), jnp.float32),
        mesh=plsc.VectorSubcoreMesh(core_axis_name="c", subcore_axis_name="s"),
        compiler_params=pltpu.CompilerParams(use_tc_tiling_on_sc=False),
        scratch_types=[
            pltpu.VMEM((4, NBR * 128), jnp.int32),
            pltpu.VMEM((4, NBR * 128), jnp.int32),
            pltpu.VMEM((4, NBR * 128), jnp.float32),
            pltpu.VMEM((3, NBR * 128, 16), jnp.float32),
            pltpu.VMEM((512, 16), jnp.float32),
            pltpu.VMEM_SHARED((N_PAD, 16), jnp.float32),
            pltpu.SemaphoreType.DMA,
            pltpu.SemaphoreType.DMA,
            pltpu.SemaphoreType.DMA,
        ],
    )(x2, srcp, dstr, wr)


def kernel(edge_index, edge_weight, users, neg_items, image_preference,
           text_preference, image_query, text_query, image_embedding,
           text_embedding, W_img, b_img, W_txt, b_txt, v_rel_mlp, t_rel_mlp,
           image_rel, text_rel, uv_agg, ut_agg):
    b_img2 = b_img.reshape(1, -1)
    b_txt2 = b_txt.reshape(1, -1)
    zeros_n = jnp.zeros((1, D), jnp.float32)

    image_emb = _mm(image_embedding, W_img, b_img2, 2000, normalize=True)
    text_emb = _mm(text_embedding, W_txt, b_txt2, 2000, normalize=True)

    x = jnp.concatenate(
        [jnp.concatenate([image_preference, image_emb], axis=0),
         jnp.concatenate([text_preference, text_emb], axis=0)], axis=1)
    x = jnp.pad(x, ((0, N_PAD - N_NODES), (0, 0)))
    x2 = x.reshape(N_PAD, FCH, 16).transpose(1, 0, 2).reshape(
        FCH * N_PAD, 16)

    src = edge_index[0]
    dst = edge_index[1]
    pad = EPAD - E
    src_p = jnp.concatenate([src, jnp.zeros((pad,), jnp.int32)])
    dst_p = jnp.concatenate([dst, jnp.zeros((pad,), jnp.int32)])
    w_p = jnp.concatenate([edge_weight[:, 0], jnp.zeros((pad,), jnp.float32)])
    srcp = (src_p.reshape(1, EPAD)
            + (jnp.arange(FCH, dtype=jnp.int32) * N_PAD).reshape(FCH, 1)
            ).reshape(FCH * EPAD)
    dstr = dst_p
    wr = w_p

    for _ in range(N_LAYERS):
        x2 = _propagate_layer(x2, srcp, dstr, wr)

    x = x2.reshape(FCH, N_PAD, 16).transpose(1, 0, 2).reshape(N_PAD, 128)
    user_preference = x[:N_USERS]
    items = x[N_USERS:N_NODES]

    comp_rel_v = _mm(image_rel, v_rel_mlp, zeros_n, 2000)
    comp_rel_t = _mm(text_rel, t_rel_mlp, zeros_n, 2000)

    image_neg_samples = jnp.concatenate(
        [uv_agg[users], image_embedding[neg_items]], axis=1)
    compressed_img_negsams = _mm(image_neg_samples, v_rel_mlp, zeros_n, 2048)
    text_neg_samples = jnp.concatenate(
        [ut_agg[users], text_embedding[neg_items]], axis=1)
    compressed_txt_negsams = _mm(text_neg_samples, t_rel_mlp, zeros_n, 2048)

    return (user_preference, items, image_query, text_query, comp_rel_v,
            comp_rel_t, compressed_img_negsams, compressed_txt_negsams,
            v_rel_mlp, t_rel_mlp, image_embedding, text_embedding)


# speedup vs baseline: 1.2115x; 1.2115x over previous
import functools

import jax
import jax.numpy as jnp
from jax import lax
from jax.experimental import pallas as pl
from jax.experimental.pallas import tpu as pltpu
from jax.experimental.pallas import tpu_sc as plsc

N_USERS = 20000
N_ITEMS = 30000
N_NODES = N_USERS + N_ITEMS
E = 800000
D = 64
DELTA = 0.8
N_LAYERS = 2

FCH = 8
FPC = FCH // 2
ER = 6400
EPAD = ER * 128
RPT = ER // 16
NBR = 8
NBATCH = RPT // NBR
N_PAD = 50048
NPT = N_PAD // 16


def _mm_body(x_ref, w_ref, b_ref, o_ref, *, normalize):
    y = jnp.dot(x_ref[...], w_ref[...], preferred_element_type=jnp.float32)
    y = y + b_ref[...]
    if normalize:
        n2 = jnp.sum(y * y, axis=1, keepdims=True)
        y = y * jax.lax.rsqrt(jnp.maximum(n2, 1e-24))
    o_ref[...] = y


def _mm(x, w, b, block_rows, normalize=False):
    m, k = x.shape
    n = w.shape[1]
    assert m % block_rows == 0
    return pl.pallas_call(
        functools.partial(_mm_body, normalize=normalize),
        grid=(m // block_rows,),
        in_specs=[
            pl.BlockSpec((block_rows, k), lambda i: (i, 0)),
            pl.BlockSpec((k, n), lambda i: (0, 0)),
            pl.BlockSpec((1, n), lambda i: (0, 0)),
        ],
        out_specs=pl.BlockSpec((block_rows, n), lambda i: (i, 0)),
        out_shape=jax.ShapeDtypeStruct((m, n), jnp.float32),
    )(x, w, b)


def _bcast_lane(vec, l):
    idx = jnp.full((16, 1), l, dtype=jnp.int32)
    dnums = lax.GatherDimensionNumbers(
        offset_dims=(), collapsed_slice_dims=(0,), start_index_map=(0,))
    return lax.gather(vec, idx, dnums, (1,),
                      mode=lax.GatherScatterMode.PROMISE_IN_BOUNDS)


def _prop_body(x2, srcp, dstr, wr, y2, sidx, didx, wv, rows, ibuf, acc,
               gsem, ssem, isem):
    c = lax.axis_index("c")
    s = lax.axis_index("s")
    row_lo = s * RPT
    node_lo = s * NPT
    EB = NBR * 128

    def fchunk_body(r, _):
        f = c * FPC + r
        fbase = f * N_PAD

        off = 0
        for sz in (512, 512, 512, 512, 512, 512, 56):
            pltpu.sync_copy(x2.at[pl.ds(fbase + node_lo + off, sz)],
                            ibuf.at[pl.ds(0, sz)])

            def init_body(i, _):
                ibuf[i] = ibuf[i] * DELTA
                return 0
            lax.fori_loop(0, sz, init_body, 0, unroll=4)
            pltpu.sync_copy(ibuf.at[pl.ds(0, sz)],
                            acc.at[pl.ds(node_lo + off, sz)])
            off += sz
        plsc.subcore_barrier()

        def fire_idx(t):
            buf = t % 4
            eb = (row_lo + t * NBR) * 128
            pltpu.async_copy(srcp.at[pl.ds(f * EPAD + eb, EB)], sidx.at[buf],
                             isem)
            pltpu.async_copy(dstr.at[pl.ds(eb, EB)], didx.at[buf], isem)
            pltpu.async_copy(wr.at[pl.ds(eb, EB)], wv.at[buf], isem)

        def drain_idx(t):
            buf = t % 4
            eb = (row_lo + t * NBR) * 128
            pltpu.make_async_copy(srcp.at[pl.ds(f * EPAD + eb, EB)],
                                  sidx.at[buf], isem).wait()
            pltpu.make_async_copy(dstr.at[pl.ds(eb, EB)], didx.at[buf],
                                  isem).wait()
            pltpu.make_async_copy(wr.at[pl.ds(eb, EB)], wv.at[buf],
                                  isem).wait()

        def fire_gathers(t):
            ib, rb = t % 4, t % 3
            for q in range(NBR):
                pltpu.async_copy(x2.at[sidx.at[ib, pl.ds(q * 128, 128)]],
                                 rows.at[rb, pl.ds(q * 128, 128)], gsem)

        def drain_gathers(t):
            ib, rb = t % 4, t % 3
            for q in range(NBR):
                pltpu.make_async_copy(x2.at[sidx.at[ib, pl.ds(q * 128, 128)]],
                                      rows.at[rb, pl.ds(q * 128, 128)],
                                      gsem).wait()

        def fire_scatter(t):
            ib, rb = t % 4, t % 3
            for q in range(NBR):
                pltpu.async_copy(rows.at[rb, pl.ds(q * 128, 128)],
                                 acc.at[didx.at[ib, pl.ds(q * 128, 128)]],
                                 ssem, add=True)

        def drain_scatter(t):
            ib, rb = t % 4, t % 3
            for q in range(NBR):
                pltpu.make_async_copy(rows.at[rb, pl.ds(q * 128, 128)],
                                      acc.at[didx.at[ib, pl.ds(q * 128, 128)]],
                                      ssem).wait()

        def compute(t):
            ib, rb = t % 4, t % 3

            @plsc.parallel_loop(0, NBR, unroll=2)
            def q_body(q):
                base = q * 128
                for k in range(8):
                    wk = wv[ib, pl.ds(base + k * 16, 16)]
                    for l in range(16):
                        i = base + k * 16 + l
                        rows[rb, i] = rows[rb, i] * _bcast_lane(wk, l)

        fire_idx(0)
        drain_idx(0)
        fire_gathers(0)
        fire_idx(1)

        def batch_body(b, _):
            @pl.when(b >= 2)
            def _():
                drain_scatter(b - 2)

            @pl.when(b + 2 < NBATCH)
            def _():
                fire_idx(b + 2)

            @pl.when(b + 1 < NBATCH)
            def _():
                drain_idx(b + 1)
                fire_gathers(b + 1)
            drain_gathers(b)
            compute(b)
            fire_scatter(b)
            return 0
        lax.fori_loop(0, NBATCH, batch_body, 0)
        drain_scatter(NBATCH - 2)
        drain_scatter(NBATCH - 1)
        plsc.subcore_barrier()

        pltpu.sync_copy(acc.at[pl.ds(node_lo, NPT)],
                        y2.at[pl.ds(fbase + node_lo, NPT)])
        plsc.subcore_barrier()
        return 0

    lax.fori_loop(0, FPC, fchunk_body, 0)


def _propagate_layer(x2, srcp, dstr, wr):
    return pl.kernel(
        _prop_body,
        out_type=jax.ShapeDtypeStruct((FCH * N_PAD, 16), jnp.float32),
        mesh=plsc.VectorSubcoreMesh(core_axis_name="c", subcore_axis_name="s"),
        compiler_params=pltpu.CompilerParams(use_tc_tiling_on_sc=False),
        scratch_types=[
            pltpu.VMEM((4, NBR * 128), jnp.int32),
            pltpu.VMEM((4, NBR * 128), jnp.int32),
            pltpu.VMEM((4, NBR * 128), jnp.float32),
            pltpu.VMEM((3, NBR * 128, 16), jnp.float32),
            pltpu.VMEM((512, 16), jnp.float32),
            pltpu.VMEM_SHARED((N_PAD, 16), jnp.float32),
            pltpu.SemaphoreType.DMA,
            pltpu.SemaphoreType.DMA,
            pltpu.SemaphoreType.DMA,
        ],
    )(x2, srcp, dstr, wr)


def kernel(edge_index, edge_weight, users, neg_items, image_preference,
           text_preference, image_query, text_query, image_embedding,
           text_embedding, W_img, b_img, W_txt, b_txt, v_rel_mlp, t_rel_mlp,
           image_rel, text_rel, uv_agg, ut_agg):
    b_img2 = b_img.reshape(1, -1)
    b_txt2 = b_txt.reshape(1, -1)
    zeros_n = jnp.zeros((1, D), jnp.float32)

    image_emb = _mm(image_embedding, W_img, b_img2, 2000, normalize=True)
    text_emb = _mm(text_embedding, W_txt, b_txt2, 2000, normalize=True)

    x = jnp.concatenate(
        [jnp.concatenate([image_preference, image_emb], axis=0),
         jnp.concatenate([text_preference, text_emb], axis=0)], axis=1)
    x = jnp.pad(x, ((0, N_PAD - N_NODES), (0, 0)))
    x2 = x.reshape(N_PAD, FCH, 16).transpose(1, 0, 2).reshape(
        FCH * N_PAD, 16)

    src = edge_index[0]
    dst = edge_index[1]
    pad = EPAD - E
    src_p = jnp.concatenate([src, jnp.zeros((pad,), jnp.int32)])
    dst_p = jnp.concatenate([dst, jnp.zeros((pad,), jnp.int32)])
    w_p = jnp.concatenate([edge_weight[:, 0], jnp.zeros((pad,), jnp.float32)])
    srcp = (src_p.reshape(1, EPAD)
            + (jnp.arange(FCH, dtype=jnp.int32) * N_PAD).reshape(FCH, 1)
            ).reshape(FCH * EPAD)
    dstr = dst_p
    wr = w_p

    for _ in range(N_LAYERS):
        x2 = _propagate_layer(x2, srcp, dstr, wr)

    x = x2.reshape(FCH, N_PAD, 16).transpose(1, 0, 2).reshape(N_PAD, 128)
    user_preference = x[:N_USERS]
    items = x[N_USERS:N_NODES]

    comp_rel_v = _mm(image_rel, v_rel_mlp, zeros_n, 2000)
    comp_rel_t = _mm(text_rel, t_rel_mlp, zeros_n, 2000)

    image_neg_samples = jnp.concatenate(
        [uv_agg[users], image_embedding[neg_items]], axis=1)
    compressed_img_negsams = _mm(image_neg_samples, v_rel_mlp, zeros_n, 2048)
    text_neg_samples = jnp.concatenate(
        [ut_agg[users], text_embedding[neg_items]], axis=1)
    compressed_txt_negsams = _mm(text_neg_samples, t_rel_mlp, zeros_n, 2048)

    return (user_preference, items, image_query, text_query, comp_rel_v,
            comp_rel_t, compressed_img_negsams, compressed_txt_negsams,
            v_rel_mlp, t_rel_mlp, image_embedding, text_embedding)


# speedup vs baseline: 1.2893x; 1.0642x over previous
import functools

import jax
import jax.numpy as jnp
from jax import lax
from jax.experimental import pallas as pl
from jax.experimental.pallas import tpu as pltpu
from jax.experimental.pallas import tpu_sc as plsc

N_USERS = 20000
N_ITEMS = 30000
N_NODES = N_USERS + N_ITEMS
E = 800000
D = 64
DELTA = 0.8
N_LAYERS = 2

FCH = 4
LW = 32
FPC = FCH // 2
ER = 6400
EPAD = ER * 128
RPT = ER // 16
NBR = 2
NBATCH = RPT // NBR
N_PAD = 50048
NPT = N_PAD // 16


def _mm_body(x_ref, w_ref, b_ref, o_ref, *, normalize):
    y = jnp.dot(x_ref[...], w_ref[...], preferred_element_type=jnp.float32)
    y = y + b_ref[...]
    if normalize:
        n2 = jnp.sum(y * y, axis=1, keepdims=True)
        y = y * jax.lax.rsqrt(jnp.maximum(n2, 1e-24))
    o_ref[...] = y


def _mm(x, w, b, block_rows, normalize=False):
    m, k = x.shape
    n = w.shape[1]
    assert m % block_rows == 0
    return pl.pallas_call(
        functools.partial(_mm_body, normalize=normalize),
        grid=(m // block_rows,),
        in_specs=[
            pl.BlockSpec((block_rows, k), lambda i: (i, 0)),
            pl.BlockSpec((k, n), lambda i: (0, 0)),
            pl.BlockSpec((1, n), lambda i: (0, 0)),
        ],
        out_specs=pl.BlockSpec((block_rows, n), lambda i: (i, 0)),
        out_shape=jax.ShapeDtypeStruct((m, n), jnp.float32),
    )(x, w, b)


def _bcast_lane(vec, l):
    idx = jnp.full((16, 1), l, dtype=jnp.int32)
    dnums = lax.GatherDimensionNumbers(
        offset_dims=(), collapsed_slice_dims=(0,), start_index_map=(0,))
    return lax.gather(vec, idx, dnums, (1,),
                      mode=lax.GatherScatterMode.PROMISE_IN_BOUNDS)


def _prop_body(x2, srcp, dstr, wr, y2, sidx, didx, wv, rows, acc,
               gsem, ssem, isem):
    c = lax.axis_index("c")
    s = lax.axis_index("s")
    row_lo = s * RPT
    node_lo = s * NPT
    EB = NBR * 128

    def fchunk_body(r, _):
        f = c * FPC + r
        fbase = f * N_PAD

        pltpu.sync_copy(x2.at[pl.ds(fbase + node_lo, NPT)],
                        acc.at[pl.ds(node_lo, NPT)])
        plsc.subcore_barrier()

        def fire_idx(t):
            buf = t % 4
            eb = (row_lo + t * NBR) * 128
            pltpu.async_copy(srcp.at[pl.ds(f * EPAD + eb, EB)], sidx.at[buf],
                             isem)
            pltpu.async_copy(dstr.at[pl.ds(eb, EB)], didx.at[buf], isem)
            pltpu.async_copy(wr.at[pl.ds(eb, EB)], wv.at[buf], isem)

        def drain_idx(t):
            buf = t % 4
            eb = (row_lo + t * NBR) * 128
            pltpu.make_async_copy(srcp.at[pl.ds(f * EPAD + eb, EB)],
                                  sidx.at[buf], isem).wait()
            pltpu.make_async_copy(dstr.at[pl.ds(eb, EB)], didx.at[buf],
                                  isem).wait()
            pltpu.make_async_copy(wr.at[pl.ds(eb, EB)], wv.at[buf],
                                  isem).wait()

        def fire_gathers(t):
            ib, rb = t % 4, t % 3
            for q in range(NBR):
                pltpu.async_copy(x2.at[sidx.at[ib, pl.ds(q * 128, 128)]],
                                 rows.at[rb, pl.ds(q * 128, 128)], gsem)

        def drain_gathers(t):
            ib, rb = t % 4, t % 3
            for q in range(NBR):
                pltpu.make_async_copy(x2.at[sidx.at[ib, pl.ds(q * 128, 128)]],
                                      rows.at[rb, pl.ds(q * 128, 128)],
                                      gsem).wait()

        def fire_scatter(t):
            ib, rb = t % 4, t % 3
            for q in range(NBR):
                pltpu.async_copy(rows.at[rb, pl.ds(q * 128, 128)],
                                 acc.at[didx.at[ib, pl.ds(q * 128, 128)]],
                                 ssem, add=True)

        def drain_scatter(t):
            ib, rb = t % 4, t % 3
            for q in range(NBR):
                pltpu.make_async_copy(rows.at[rb, pl.ds(q * 128, 128)],
                                      acc.at[didx.at[ib, pl.ds(q * 128, 128)]],
                                      ssem).wait()

        def compute(t):
            ib, rb = t % 4, t % 3

            @plsc.parallel_loop(0, EB // 16, unroll=2)
            def k_body(k):
                wk = wv[ib, pl.ds(k * 16, 16)]
                for l in range(16):
                    i = k * 16 + l
                    wl = _bcast_lane(wk, l)
                    rows[rb, i, pl.ds(0, 16)] = rows[rb, i, pl.ds(0, 16)] * wl
                    rows[rb, i, pl.ds(16, 16)] = (
                        rows[rb, i, pl.ds(16, 16)] * wl)

        fire_idx(0)
        drain_idx(0)
        fire_gathers(0)
        fire_idx(1)

        def batch_body(b, _):
            @pl.when(b >= 2)
            def _():
                drain_scatter(b - 2)

            @pl.when(b + 2 < NBATCH)
            def _():
                fire_idx(b + 2)

            @pl.when(b + 1 < NBATCH)
            def _():
                drain_idx(b + 1)
                fire_gathers(b + 1)
            drain_gathers(b)
            compute(b)
            fire_scatter(b)
            return 0
        lax.fori_loop(0, NBATCH, batch_body, 0)
        drain_scatter(NBATCH - 2)
        drain_scatter(NBATCH - 1)
        plsc.subcore_barrier()

        pltpu.sync_copy(acc.at[pl.ds(node_lo, NPT)],
                        y2.at[pl.ds(fbase + node_lo, NPT)])
        plsc.subcore_barrier()
        return 0

    lax.fori_loop(0, FPC, fchunk_body, 0)


def _propagate_layer(x2, srcp, dstr, wr):
    return pl.kernel(
        _prop_body,
        out_type=jax.ShapeDtypeStruct((FCH * N_PAD, LW), jnp.float32),
        mesh=plsc.VectorSubcoreMesh(core_axis_name="c", subcore_axis_name="s"),
        compiler_params=pltpu.CompilerParams(use_tc_tiling_on_sc=False),
        scratch_types=[
            pltpu.VMEM((4, NBR * 128), jnp.int32),
            pltpu.VMEM((4, NBR * 128), jnp.int32),
            pltpu.VMEM((4, NBR * 128), jnp.float32),
            pltpu.VMEM((3, NBR * 128, LW), jnp.float32),
            pltpu.VMEM_SHARED((N_PAD, LW), jnp.float32),
            pltpu.SemaphoreType.DMA,
            pltpu.SemaphoreType.DMA,
            pltpu.SemaphoreType.DMA,
        ],
    )(x2, srcp, dstr, wr)


def kernel(edge_index, edge_weight, users, neg_items, image_preference,
           text_preference, image_query, text_query, image_embedding,
           text_embedding, W_img, b_img, W_txt, b_txt, v_rel_mlp, t_rel_mlp,
           image_rel, text_rel, uv_agg, ut_agg):
    b_img2 = b_img.reshape(1, -1)
    b_txt2 = b_txt.reshape(1, -1)
    zeros_n = jnp.zeros((1, D), jnp.float32)

    image_emb = _mm(image_embedding, W_img, b_img2, 2000, normalize=True)
    text_emb = _mm(text_embedding, W_txt, b_txt2, 2000, normalize=True)

    x = jnp.concatenate(
        [jnp.concatenate([image_preference, image_emb], axis=0),
         jnp.concatenate([text_preference, text_emb], axis=0)], axis=1)
    x = jnp.pad(x, ((0, N_PAD - N_NODES), (0, 0)))
    x2 = x.reshape(N_PAD, FCH, LW).transpose(1, 0, 2).reshape(
        FCH * N_PAD, LW)

    src = edge_index[0]
    dst = edge_index[1]
    pad = EPAD - E
    src_p = jnp.concatenate([src, jnp.zeros((pad,), jnp.int32)])
    dst_p = jnp.concatenate([dst, jnp.zeros((pad,), jnp.int32)])
    w_p = jnp.concatenate([edge_weight[:, 0], jnp.zeros((pad,), jnp.float32)])
    srcp = (src_p.reshape(1, EPAD)
            + (jnp.arange(FCH, dtype=jnp.int32) * N_PAD).reshape(FCH, 1)
            ).reshape(FCH * EPAD)
    dstr = dst_p
    wr = w_p

    for _ in range(N_LAYERS):
        y2 = _propagate_layer(x2, srcp, dstr, wr)
        x2 = y2 - (1.0 - DELTA) * x2

    x = x2.reshape(FCH, N_PAD, LW).transpose(1, 0, 2).reshape(N_PAD, 128)
    user_preference = x[:N_USERS]
    items = x[N_USERS:N_NODES]

    comp_rel_v = _mm(image_rel, v_rel_mlp, zeros_n, 2000)
    comp_rel_t = _mm(text_rel, t_rel_mlp, zeros_n, 2000)

    image_neg_samples = jnp.concatenate(
        [uv_agg[users], image_embedding[neg_items]], axis=1)
    compressed_img_negsams = _mm(image_neg_samples, v_rel_mlp, zeros_n, 2048)
    text_neg_samples = jnp.concatenate(
        [ut_agg[users], text_embedding[neg_items]], axis=1)
    compressed_txt_negsams = _mm(text_neg_samples, t_rel_mlp, zeros_n, 2048)

    return (user_preference, items, image_query, text_query, comp_rel_v,
            comp_rel_t, compressed_img_negsams, compressed_txt_negsams,
            v_rel_mlp, t_rel_mlp, image_embedding, text_embedding)


# speedup vs baseline: 1.3311x; 1.0324x over previous
import functools

import jax
import jax.numpy as jnp
from jax import lax
from jax.experimental import pallas as pl
from jax.experimental.pallas import tpu as pltpu
from jax.experimental.pallas import tpu_sc as plsc

N_USERS = 20000
N_ITEMS = 30000
N_NODES = N_USERS + N_ITEMS
E = 800000
D = 64
DELTA = 0.8
N_LAYERS = 2

FCH = 4
LW = 32
FPC = FCH // 2
ER = 6400
EPAD = ER * 128
RPT = ER // 16
NBR = 2
NBATCH = RPT // NBR
N_PAD = 50048
NPT = N_PAD // 16


def _mm_body(x_ref, w_ref, b_ref, o_ref, *, normalize):
    y = jnp.dot(x_ref[...], w_ref[...], preferred_element_type=jnp.float32)
    y = y + b_ref[...]
    if normalize:
        n2 = jnp.sum(y * y, axis=1, keepdims=True)
        y = y * jax.lax.rsqrt(jnp.maximum(n2, 1e-24))
    o_ref[...] = y


def _mm(x, w, b, block_rows, normalize=False):
    m, k = x.shape
    n = w.shape[1]
    assert m % block_rows == 0
    return pl.pallas_call(
        functools.partial(_mm_body, normalize=normalize),
        grid=(m // block_rows,),
        in_specs=[
            pl.BlockSpec((block_rows, k), lambda i: (i, 0)),
            pl.BlockSpec((k, n), lambda i: (0, 0)),
            pl.BlockSpec((1, n), lambda i: (0, 0)),
        ],
        out_specs=pl.BlockSpec((block_rows, n), lambda i: (i, 0)),
        out_shape=jax.ShapeDtypeStruct((m, n), jnp.float32),
    )(x, w, b)


def _bcast_lane(vec, l):
    idx = jnp.full((16, 1), l, dtype=jnp.int32)
    dnums = lax.GatherDimensionNumbers(
        offset_dims=(), collapsed_slice_dims=(0,), start_index_map=(0,))
    return lax.gather(vec, idx, dnums, (1,),
                      mode=lax.GatherScatterMode.PROMISE_IN_BOUNDS)


def _prop_body(x2, zer, srcp, dstr, wr, y2, sidx, didx, wv, rows_bf,
               rows32, acc, gsem, ssem, isem):
    c = lax.axis_index("c")
    s = lax.axis_index("s")
    row_lo = s * RPT
    node_lo = s * NPT
    EB = NBR * 128

    def fchunk_body(r, _):
        f = c * FPC + r
        fbase = f * N_PAD

        pltpu.sync_copy(zer.at[pl.ds(node_lo, NPT)],
                        acc.at[pl.ds(node_lo, NPT)])
        plsc.subcore_barrier()

        def fire_idx(t):
            eb = (row_lo + t * NBR) * 128
            pltpu.async_copy(srcp.at[pl.ds(f * EPAD + eb, EB)],
                             sidx.at[t % 2], isem)
            pltpu.async_copy(dstr.at[pl.ds(eb, EB)], didx.at[t % 4], isem)
            pltpu.async_copy(wr.at[pl.ds(eb, EB)], wv.at[t % 3], isem)

        def drain_idx(t):
            eb = (row_lo + t * NBR) * 128
            pltpu.make_async_copy(srcp.at[pl.ds(f * EPAD + eb, EB)],
                                  sidx.at[t % 2], isem).wait()
            pltpu.make_async_copy(dstr.at[pl.ds(eb, EB)], didx.at[t % 4],
                                  isem).wait()
            pltpu.make_async_copy(wr.at[pl.ds(eb, EB)], wv.at[t % 3],
                                  isem).wait()

        def fire_gathers(t):
            for q in range(NBR):
                pltpu.async_copy(x2.at[sidx.at[t % 2, pl.ds(q * 128, 128)]],
                                 rows_bf.at[t % 2, pl.ds(q * 128, 128)], gsem)

        def drain_gathers(t):
            for q in range(NBR):
                pltpu.make_async_copy(
                    x2.at[sidx.at[t % 2, pl.ds(q * 128, 128)]],
                    rows_bf.at[t % 2, pl.ds(q * 128, 128)], gsem).wait()

        def fire_scatter(t):
            for q in range(NBR):
                pltpu.async_copy(rows32.at[t % 2, pl.ds(q * 128, 128)],
                                 acc.at[didx.at[t % 4, pl.ds(q * 128, 128)]],
                                 ssem, add=True)

        def drain_scatter(t):
            for q in range(NBR):
                pltpu.make_async_copy(
                    rows32.at[t % 2, pl.ds(q * 128, 128)],
                    acc.at[didx.at[t % 4, pl.ds(q * 128, 128)]], ssem).wait()

        def compute(t):
            ib3, ib2 = t % 3, t % 2

            @plsc.parallel_loop(0, EB // 16, unroll=2)
            def k_body(k):
                wk = wv[ib3, pl.ds(k * 16, 16)]
                for l in range(16):
                    i = k * 16 + l
                    wl = _bcast_lane(wk, l)
                    a, b2 = plsc.unpack(rows_bf[ib2, i],
                                        format=plsc.PackFormat.INTERLEAVED)
                    rows32[ib2, i, pl.ds(0, 16)] = a * wl
                    rows32[ib2, i, pl.ds(16, 16)] = b2 * wl

        fire_idx(0)
        drain_idx(0)
        fire_gathers(0)
        fire_idx(1)

        def batch_body(b, _):
            drain_gathers(b)

            @pl.when(b >= 2)
            def _():
                drain_scatter(b - 2)

            @pl.when(b + 2 < NBATCH)
            def _():
                fire_idx(b + 2)

            @pl.when(b + 1 < NBATCH)
            def _():
                drain_idx(b + 1)
                fire_gathers(b + 1)
            compute(b)
            fire_scatter(b)
            return 0
        lax.fori_loop(0, NBATCH, batch_body, 0)
        drain_scatter(NBATCH - 2)
        drain_scatter(NBATCH - 1)
        plsc.subcore_barrier()

        pltpu.sync_copy(acc.at[pl.ds(node_lo, NPT)],
                        y2.at[pl.ds(fbase + node_lo, NPT)])
        plsc.subcore_barrier()
        return 0

    lax.fori_loop(0, FPC, fchunk_body, 0)


def _propagate_layer(x2b, zer, srcp, dstr, wr):
    return pl.kernel(
        _prop_body,
        out_type=jax.ShapeDtypeStruct((FCH * N_PAD, LW), jnp.float32),
        mesh=plsc.VectorSubcoreMesh(core_axis_name="c", subcore_axis_name="s"),
        compiler_params=pltpu.CompilerParams(use_tc_tiling_on_sc=False,
                                             needs_layout_passes=False),
        scratch_types=[
            pltpu.VMEM((2, NBR * 128), jnp.int32),
            pltpu.VMEM((4, NBR * 128), jnp.int32),
            pltpu.VMEM((3, NBR * 128), jnp.float32),
            pltpu.VMEM((2, NBR * 128, LW), jnp.bfloat16),
            pltpu.VMEM((2, NBR * 128, LW), jnp.float32),
            pltpu.VMEM_SHARED((N_PAD, LW), jnp.float32),
            pltpu.SemaphoreType.DMA,
            pltpu.SemaphoreType.DMA,
            pltpu.SemaphoreType.DMA,
        ],
    )(x2b, zer, srcp, dstr, wr)


def kernel(edge_index, edge_weight, users, neg_items, image_preference,
           text_preference, image_query, text_query, image_embedding,
           text_embedding, W_img, b_img, W_txt, b_txt, v_rel_mlp, t_rel_mlp,
           image_rel, text_rel, uv_agg, ut_agg):
    b_img2 = b_img.reshape(1, -1)
    b_txt2 = b_txt.reshape(1, -1)
    zeros_n = jnp.zeros((1, D), jnp.float32)

    image_emb = _mm(image_embedding, W_img, b_img2, 2000, normalize=True)
    text_emb = _mm(text_embedding, W_txt, b_txt2, 2000, normalize=True)

    x = jnp.concatenate(
        [jnp.concatenate([image_preference, image_emb], axis=0),
         jnp.concatenate([text_preference, text_emb], axis=0)], axis=1)
    x = jnp.pad(x, ((0, N_PAD - N_NODES), (0, 0)))
    xn = x.reshape(N_PAD, FCH, LW).transpose(1, 0, 2).reshape(
        FCH * N_PAD, LW)
    zer = jnp.zeros((N_PAD, LW), jnp.float32)

    src = edge_index[0]
    dst = edge_index[1]
    pad = EPAD - E
    src_p = jnp.concatenate([src, jnp.zeros((pad,), jnp.int32)])
    dst_p = jnp.concatenate([dst, jnp.zeros((pad,), jnp.int32)])
    w_p = jnp.concatenate([edge_weight[:, 0], jnp.zeros((pad,), jnp.float32)])
    srcp = (src_p.reshape(1, EPAD)
            + (jnp.arange(FCH, dtype=jnp.int32) * N_PAD).reshape(FCH, 1)
            ).reshape(FCH * EPAD)
    dstr = dst_p
    wr = w_p

    for _ in range(N_LAYERS):
        x2b = xn.reshape(-1, 2, 16).transpose(0, 2, 1).reshape(
            -1, LW).astype(jnp.bfloat16)
        y2 = _propagate_layer(x2b, zer, srcp, dstr, wr)
        xn = y2 + DELTA * xn

    x = xn.reshape(FCH, N_PAD, LW).transpose(1, 0, 2).reshape(N_PAD, 128)
    user_preference = x[:N_USERS]
    items = x[N_USERS:N_NODES]

    comp_rel_v = _mm(image_rel, v_rel_mlp, zeros_n, 2000)
    comp_rel_t = _mm(text_rel, t_rel_mlp, zeros_n, 2000)

    image_neg_samples = jnp.concatenate(
        [uv_agg[users], image_embedding[neg_items]], axis=1)
    compressed_img_negsams = _mm(image_neg_samples, v_rel_mlp, zeros_n, 2048)
    text_neg_samples = jnp.concatenate(
        [ut_agg[users], text_embedding[neg_items]], axis=1)
    compressed_txt_negsams = _mm(text_neg_samples, t_rel_mlp, zeros_n, 2048)

    return (user_preference, items, image_query, text_query, comp_rel_v,
            comp_rel_t, compressed_img_negsams, compressed_txt_negsams,
            v_rel_mlp, t_rel_mlp, image_embedding, text_embedding)


# speedup vs baseline: 1.4427x; 1.0838x over previous
import functools

import jax
import jax.numpy as jnp
from jax import lax
from jax.experimental import pallas as pl
from jax.experimental.pallas import tpu as pltpu
from jax.experimental.pallas import tpu_sc as plsc

N_USERS = 20000
N_ITEMS = 30000
N_NODES = N_USERS + N_ITEMS
E = 800000
D = 64
DELTA = 0.8
N_LAYERS = 2

FCH = 4
LW = 32
FPC = FCH // 2
ER = 6400
EPAD = ER * 128
RPT = ER // 16
NBR = 2
NBATCH = RPT // NBR
N_PAD = 50048
NPT = N_PAD // 16


def _mm_body(x_ref, w_ref, b_ref, o_ref, *, normalize):
    y = jnp.dot(x_ref[...], w_ref[...], preferred_element_type=jnp.float32)
    y = y + b_ref[...]
    if normalize:
        n2 = jnp.sum(y * y, axis=1, keepdims=True)
        y = y * jax.lax.rsqrt(jnp.maximum(n2, 1e-24))
    o_ref[...] = y


def _mm(x, w, b, block_rows, normalize=False):
    m, k = x.shape
    n = w.shape[1]
    assert m % block_rows == 0
    return pl.pallas_call(
        functools.partial(_mm_body, normalize=normalize),
        grid=(m // block_rows,),
        in_specs=[
            pl.BlockSpec((block_rows, k), lambda i: (i, 0)),
            pl.BlockSpec((k, n), lambda i: (0, 0)),
            pl.BlockSpec((1, n), lambda i: (0, 0)),
        ],
        out_specs=pl.BlockSpec((block_rows, n), lambda i: (i, 0)),
        out_shape=jax.ShapeDtypeStruct((m, n), jnp.float32),
    )(x, w, b)


def _bcast_lane(vec, l):
    idx = jnp.full((16, 1), l, dtype=jnp.int32)
    dnums = lax.GatherDimensionNumbers(
        offset_dims=(), collapsed_slice_dims=(0,), start_index_map=(0,))
    return lax.gather(vec, idx, dnums, (1,),
                      mode=lax.GatherScatterMode.PROMISE_IN_BOUNDS)


def _prop_body(x2, zer, srcp, dstr, wr, y2, sidx, didx, wv, rows_bf,
               rows32, acc, gsem, ssem, isem, jsem):
    c = lax.axis_index("c")
    s = lax.axis_index("s")
    row_lo = s * RPT
    node_lo = s * NPT
    EB = NBR * 128

    def fchunk_body(r, _):
        f = c * FPC + r
        fbase = f * N_PAD

        pltpu.sync_copy(zer.at[pl.ds(node_lo, NPT)],
                        acc.at[pl.ds(node_lo, NPT)])
        plsc.subcore_barrier()

        def fire_idx_s(t):
            eb = (row_lo + t * NBR) * 128
            pltpu.async_copy(srcp.at[pl.ds(f * EPAD + eb, EB)],
                             sidx.at[t % 3], isem)

        def drain_idx_s(t):
            eb = (row_lo + t * NBR) * 128
            pltpu.make_async_copy(srcp.at[pl.ds(f * EPAD + eb, EB)],
                                  sidx.at[t % 3], isem).wait()

        def fire_idx_dw(t):
            eb = (row_lo + t * NBR) * 128
            pltpu.async_copy(dstr.at[pl.ds(eb, EB)], didx.at[t % 3], jsem)
            pltpu.async_copy(wr.at[pl.ds(eb, EB)], wv.at[t % 2], jsem)

        def drain_idx_dw(t):
            eb = (row_lo + t * NBR) * 128
            pltpu.make_async_copy(dstr.at[pl.ds(eb, EB)], didx.at[t % 3],
                                  jsem).wait()
            pltpu.make_async_copy(wr.at[pl.ds(eb, EB)], wv.at[t % 2],
                                  jsem).wait()

        def fire_gathers(t):
            for q in range(NBR):
                pltpu.async_copy(x2.at[sidx.at[t % 3, pl.ds(q * 128, 128)]],
                                 rows_bf.at[t % 3, pl.ds(q * 128, 128)], gsem)

        def drain_gathers(t):
            for q in range(NBR):
                pltpu.make_async_copy(
                    x2.at[sidx.at[t % 3, pl.ds(q * 128, 128)]],
                    rows_bf.at[t % 3, pl.ds(q * 128, 128)], gsem).wait()

        def fire_scatter(t):
            for q in range(NBR):
                pltpu.async_copy(rows32.at[t % 2, pl.ds(q * 128, 128)],
                                 acc.at[didx.at[t % 3, pl.ds(q * 128, 128)]],
                                 ssem, add=True)

        def drain_scatter(t):
            for q in range(NBR):
                pltpu.make_async_copy(
                    rows32.at[t % 2, pl.ds(q * 128, 128)],
                    acc.at[didx.at[t % 3, pl.ds(q * 128, 128)]], ssem).wait()

        def compute(t):
            ib3, ib2 = t % 3, t % 2

            @plsc.parallel_loop(0, EB // 16, unroll=2)
            def k_body(k):
                wk = wv[t % 2, pl.ds(k * 16, 16)]
                for l in range(16):
                    i = k * 16 + l
                    wl = _bcast_lane(wk, l)
                    a, b2 = plsc.unpack(rows_bf[ib3, i],
                                        format=plsc.PackFormat.INTERLEAVED)
                    rows32[ib2, i, pl.ds(0, 16)] = a * wl
                    rows32[ib2, i, pl.ds(16, 16)] = b2 * wl

        fire_idx_s(0)
        fire_idx_s(1)
        fire_idx_s(2)
        drain_idx_s(0)
        fire_gathers(0)
        drain_idx_s(1)
        fire_gathers(1)
        fire_idx_dw(0)

        def batch_body(b, _):
            drain_gathers(b)

            @pl.when(b >= 2)
            def _():
                drain_scatter(b - 2)

            @pl.when(b + 3 < NBATCH)
            def _():
                fire_idx_s(b + 3)

            @pl.when(b + 1 < NBATCH)
            def _():
                fire_idx_dw(b + 1)

            @pl.when(b + 2 < NBATCH)
            def _():
                drain_idx_s(b + 2)
                fire_gathers(b + 2)
            drain_idx_dw(b)
            compute(b)
            fire_scatter(b)
            return 0
        lax.fori_loop(0, NBATCH, batch_body, 0)
        drain_scatter(NBATCH - 2)
        drain_scatter(NBATCH - 1)
        plsc.subcore_barrier()

        pltpu.sync_copy(acc.at[pl.ds(node_lo, NPT)],
                        y2.at[pl.ds(fbase + node_lo, NPT)])
        plsc.subcore_barrier()
        return 0

    lax.fori_loop(0, FPC, fchunk_body, 0)


def _propagate_layer(x2b, zer, srcp, dstr, wr):
    return pl.kernel(
        _prop_body,
        out_type=jax.ShapeDtypeStruct((FCH * N_PAD, LW), jnp.float32),
        mesh=plsc.VectorSubcoreMesh(core_axis_name="c", subcore_axis_name="s"),
        compiler_params=pltpu.CompilerParams(use_tc_tiling_on_sc=False,
                                             needs_layout_passes=False),
        scratch_types=[
            pltpu.VMEM((3, NBR * 128), jnp.int32),
            pltpu.VMEM((3, NBR * 128), jnp.int32),
            pltpu.VMEM((2, NBR * 128), jnp.float32),
            pltpu.VMEM((3, NBR * 128, LW), jnp.bfloat16),
            pltpu.VMEM((2, NBR * 128, LW), jnp.float32),
            pltpu.VMEM_SHARED((N_PAD, LW), jnp.float32),
            pltpu.SemaphoreType.DMA,
            pltpu.SemaphoreType.DMA,
            pltpu.SemaphoreType.DMA,
            pltpu.SemaphoreType.DMA,
        ],
    )(x2b, zer, srcp, dstr, wr)


def kernel(edge_index, edge_weight, users, neg_items, image_preference,
           text_preference, image_query, text_query, image_embedding,
           text_embedding, W_img, b_img, W_txt, b_txt, v_rel_mlp, t_rel_mlp,
           image_rel, text_rel, uv_agg, ut_agg):
    b_img2 = b_img.reshape(1, -1)
    b_txt2 = b_txt.reshape(1, -1)
    zeros_n = jnp.zeros((1, D), jnp.float32)

    image_emb = _mm(image_embedding, W_img, b_img2, 2000, normalize=True)
    text_emb = _mm(text_embedding, W_txt, b_txt2, 2000, normalize=True)

    x = jnp.concatenate(
        [jnp.concatenate([image_preference, image_emb], axis=0),
         jnp.concatenate([text_preference, text_emb], axis=0)], axis=1)
    x = jnp.pad(x, ((0, N_PAD - N_NODES), (0, 0)))
    xn = x.reshape(N_PAD, FCH, LW).transpose(1, 0, 2).reshape(
        FCH * N_PAD, LW)
    zer = jnp.zeros((N_PAD, LW), jnp.float32)

    src = edge_index[0]
    dst = edge_index[1]
    pad = EPAD - E
    src_p = jnp.concatenate([src, jnp.zeros((pad,), jnp.int32)])
    dst_p = jnp.concatenate([dst, jnp.zeros((pad,), jnp.int32)])
    w_p = jnp.concatenate([edge_weight[:, 0], jnp.zeros((pad,), jnp.float32)])
    srcp = (src_p.reshape(1, EPAD)
            + (jnp.arange(FCH, dtype=jnp.int32) * N_PAD).reshape(FCH, 1)
            ).reshape(FCH * EPAD)
    dstr = dst_p
    wr = w_p

    for _ in range(N_LAYERS):
        x2b = xn.reshape(-1, 2, 16).transpose(0, 2, 1).reshape(
            -1, LW).astype(jnp.bfloat16)
        y2 = _propagate_layer(x2b, zer, srcp, dstr, wr)
        xn = y2 + DELTA * xn

    x = xn.reshape(FCH, N_PAD, LW).transpose(1, 0, 2).reshape(N_PAD, 128)
    user_preference = x[:N_USERS]
    items = x[N_USERS:N_NODES]

    comp_rel_v = _mm(image_rel, v_rel_mlp, zeros_n, 2000)
    comp_rel_t = _mm(text_rel, t_rel_mlp, zeros_n, 2000)

    image_neg_samples = jnp.concatenate(
        [uv_agg[users], image_embedding[neg_items]], axis=1)
    compressed_img_negsams = _mm(image_neg_samples, v_rel_mlp, zeros_n, 2048)
    text_neg_samples = jnp.concatenate(
        [ut_agg[users], text_embedding[neg_items]], axis=1)
    compressed_txt_negsams = _mm(text_neg_samples, t_rel_mlp, zeros_n, 2048)

    return (user_preference, items, image_query, text_query, comp_rel_v,
            comp_rel_t, compressed_img_negsams, compressed_txt_negsams,
            v_rel_mlp, t_rel_mlp, image_embedding, text_embedding)


# speedup vs baseline: 1.4448x; 1.0014x over previous
import functools

import jax
import jax.numpy as jnp
from jax import lax
from jax.experimental import pallas as pl
from jax.experimental.pallas import tpu as pltpu
from jax.experimental.pallas import tpu_sc as plsc

N_USERS = 20000
N_ITEMS = 30000
N_NODES = N_USERS + N_ITEMS
E = 800000
D = 64
DELTA = 0.8
N_LAYERS = 2

FCH = 4
LW = 32
FPC = FCH // 2
ER = 6400
EPAD = ER * 128
RPT = ER // 16
NBR = 2
NBATCH = RPT // NBR
N_PAD = 50048
NPT = N_PAD // 16


def _mm_body(x_ref, w_ref, b_ref, o_ref, *, normalize):
    y = jnp.dot(x_ref[...].astype(jnp.bfloat16),
                w_ref[...].astype(jnp.bfloat16),
                preferred_element_type=jnp.float32)
    y = y + b_ref[...]
    if normalize:
        n2 = jnp.sum(y * y, axis=1, keepdims=True)
        y = y * jax.lax.rsqrt(jnp.maximum(n2, 1e-24))
    o_ref[...] = y


def _mm(x, w, b, block_rows, normalize=False):
    m, k = x.shape
    n = w.shape[1]
    assert m % block_rows == 0
    return pl.pallas_call(
        functools.partial(_mm_body, normalize=normalize),
        grid=(m // block_rows,),
        in_specs=[
            pl.BlockSpec((block_rows, k), lambda i: (i, 0)),
            pl.BlockSpec((k, n), lambda i: (0, 0)),
            pl.BlockSpec((1, n), lambda i: (0, 0)),
        ],
        out_specs=pl.BlockSpec((block_rows, n), lambda i: (i, 0)),
        out_shape=jax.ShapeDtypeStruct((m, n), jnp.float32),
    )(x, w, b)


def _bcast_lane(vec, l):
    idx = jnp.full((16, 1), l, dtype=jnp.int32)
    dnums = lax.GatherDimensionNumbers(
        offset_dims=(), collapsed_slice_dims=(0,), start_index_map=(0,))
    return lax.gather(vec, idx, dnums, (1,),
                      mode=lax.GatherScatterMode.PROMISE_IN_BOUNDS)


def _prop_body(x2, zer, srcp, dstr, wr, y2, sidx, didx, wv, rows_bf,
               rows32, acc, gsem, ssem, isem, jsem):
    c = lax.axis_index("c")
    s = lax.axis_index("s")
    row_lo = s * RPT
    node_lo = s * NPT
    EB = NBR * 128

    def fchunk_body(r, _):
        f = c * FPC + r
        fbase = f * N_PAD

        pltpu.sync_copy(zer.at[pl.ds(node_lo, NPT)],
                        acc.at[pl.ds(node_lo, NPT)])
        plsc.subcore_barrier()

        def fire_idx_s(t):
            eb = (row_lo + t * NBR) * 128
            pltpu.async_copy(srcp.at[pl.ds(f * EPAD + eb, EB)],
                             sidx.at[t % 3], isem)

        def drain_idx_s(t):
            eb = (row_lo + t * NBR) * 128
            pltpu.make_async_copy(srcp.at[pl.ds(f * EPAD + eb, EB)],
                                  sidx.at[t % 3], isem).wait()

        def fire_idx_dw(t):
            eb = (row_lo + t * NBR) * 128
            pltpu.async_copy(dstr.at[pl.ds(eb, EB)], didx.at[t % 3], jsem)
            pltpu.async_copy(wr.at[pl.ds(eb, EB)], wv.at[t % 2], jsem)

        def drain_idx_dw(t):
            eb = (row_lo + t * NBR) * 128
            pltpu.make_async_copy(dstr.at[pl.ds(eb, EB)], didx.at[t % 3],
                                  jsem).wait()
            pltpu.make_async_copy(wr.at[pl.ds(eb, EB)], wv.at[t % 2],
                                  jsem).wait()

        def fire_gathers(t):
            for q in range(NBR):
                pltpu.async_copy(x2.at[sidx.at[t % 3, pl.ds(q * 128, 128)]],
                                 rows_bf.at[t % 3, pl.ds(q * 128, 128)], gsem)

        def drain_gathers(t):
            for q in range(NBR):
                pltpu.make_async_copy(
                    x2.at[sidx.at[t % 3, pl.ds(q * 128, 128)]],
                    rows_bf.at[t % 3, pl.ds(q * 128, 128)], gsem).wait()

        def fire_scatter(t):
            for q in range(NBR):
                pltpu.async_copy(rows32.at[t % 2, pl.ds(q * 128, 128)],
                                 acc.at[didx.at[t % 3, pl.ds(q * 128, 128)]],
                                 ssem, add=True)

        def drain_scatter(t):
            for q in range(NBR):
                pltpu.make_async_copy(
                    rows32.at[t % 2, pl.ds(q * 128, 128)],
                    acc.at[didx.at[t % 3, pl.ds(q * 128, 128)]], ssem).wait()

        def compute(t):
            ib3, ib2 = t % 3, t % 2

            @plsc.parallel_loop(0, EB // 16, unroll=2)
            def k_body(k):
                wk = wv[t % 2, pl.ds(k * 16, 16)]
                for l in range(16):
                    i = k * 16 + l
                    wl = _bcast_lane(wk, l)
                    a, b2 = plsc.unpack(rows_bf[ib3, i],
                                        format=plsc.PackFormat.INTERLEAVED)
                    rows32[ib2, i, pl.ds(0, 16)] = a * wl
                    rows32[ib2, i, pl.ds(16, 16)] = b2 * wl

        fire_idx_s(0)
        fire_idx_s(1)
        fire_idx_s(2)
        drain_idx_s(0)
        fire_gathers(0)
        drain_idx_s(1)
        fire_gathers(1)
        fire_idx_dw(0)

        def batch_body(b, _):
            drain_gathers(b)

            @pl.when(b >= 2)
            def _():
                drain_scatter(b - 2)

            @pl.when(b + 3 < NBATCH)
            def _():
                fire_idx_s(b + 3)

            @pl.when(b + 1 < NBATCH)
            def _():
                fire_idx_dw(b + 1)

            @pl.when(b + 2 < NBATCH)
            def _():
                drain_idx_s(b + 2)
                fire_gathers(b + 2)
            drain_idx_dw(b)
            compute(b)
            fire_scatter(b)
            return 0
        lax.fori_loop(0, NBATCH, batch_body, 0)
        drain_scatter(NBATCH - 2)
        drain_scatter(NBATCH - 1)
        plsc.subcore_barrier()

        pltpu.sync_copy(acc.at[pl.ds(node_lo, NPT)],
                        y2.at[pl.ds(fbase + node_lo, NPT)])
        plsc.subcore_barrier()
        return 0

    lax.fori_loop(0, FPC, fchunk_body, 0)


def _propagate_layer(x2b, zer, srcp, dstr, wr):
    return pl.kernel(
        _prop_body,
        out_type=jax.ShapeDtypeStruct((FCH * N_PAD, LW), jnp.float32),
        mesh=plsc.VectorSubcoreMesh(core_axis_name="c", subcore_axis_name="s"),
        compiler_params=pltpu.CompilerParams(use_tc_tiling_on_sc=False,
                                             needs_layout_passes=False),
        scratch_types=[
            pltpu.VMEM((3, NBR * 128), jnp.int32),
            pltpu.VMEM((3, NBR * 128), jnp.int32),
            pltpu.VMEM((2, NBR * 128), jnp.float32),
            pltpu.VMEM((3, NBR * 128, LW), jnp.bfloat16),
            pltpu.VMEM((2, NBR * 128, LW), jnp.float32),
            pltpu.VMEM_SHARED((N_PAD, LW), jnp.float32),
            pltpu.SemaphoreType.DMA,
            pltpu.SemaphoreType.DMA,
            pltpu.SemaphoreType.DMA,
            pltpu.SemaphoreType.DMA,
        ],
    )(x2b, zer, srcp, dstr, wr)


def kernel(edge_index, edge_weight, users, neg_items, image_preference,
           text_preference, image_query, text_query, image_embedding,
           text_embedding, W_img, b_img, W_txt, b_txt, v_rel_mlp, t_rel_mlp,
           image_rel, text_rel, uv_agg, ut_agg):
    b_img2 = b_img.reshape(1, -1)
    b_txt2 = b_txt.reshape(1, -1)
    zeros_n = jnp.zeros((1, D), jnp.float32)

    image_emb = _mm(image_embedding, W_img, b_img2, 2000, normalize=True)
    text_emb = _mm(text_embedding, W_txt, b_txt2, 2000, normalize=True)

    x = jnp.concatenate(
        [jnp.concatenate([image_preference, image_emb], axis=0),
         jnp.concatenate([text_preference, text_emb], axis=0)], axis=1)
    x = jnp.pad(x, ((0, N_PAD - N_NODES), (0, 0)))
    xn = x.reshape(N_PAD, FCH, LW).transpose(1, 0, 2).reshape(
        FCH * N_PAD, LW)
    zer = jnp.zeros((N_PAD, LW), jnp.float32)

    src = edge_index[0]
    dst = edge_index[1]
    pad = EPAD - E
    src_p = jnp.concatenate([src, jnp.zeros((pad,), jnp.int32)])
    dst_p = jnp.concatenate([dst, jnp.zeros((pad,), jnp.int32)])
    w_p = jnp.concatenate([edge_weight[:, 0], jnp.zeros((pad,), jnp.float32)])
    srcp = (src_p.reshape(1, EPAD)
            + (jnp.arange(FCH, dtype=jnp.int32) * N_PAD).reshape(FCH, 1)
            ).reshape(FCH * EPAD)
    dstr = dst_p
    wr = w_p

    for _ in range(N_LAYERS):
        x2b = xn.reshape(-1, 2, 16).transpose(0, 2, 1).reshape(
            -1, LW).astype(jnp.bfloat16)
        y2 = _propagate_layer(x2b, zer, srcp, dstr, wr)
        xn = y2 + DELTA * xn

    x = xn.reshape(FCH, N_PAD, LW).transpose(1, 0, 2).reshape(N_PAD, 128)
    user_preference = x[:N_USERS]
    items = x[N_USERS:N_NODES]

    comp_rel_v = _mm(image_rel, v_rel_mlp, zeros_n, 2000)
    comp_rel_t = _mm(text_rel, t_rel_mlp, zeros_n, 2000)

    image_neg_samples = jnp.concatenate(
        [uv_agg[users], image_embedding[neg_items]], axis=1)
    compressed_img_negsams = _mm(image_neg_samples, v_rel_mlp, zeros_n, 2048)
    text_neg_samples = jnp.concatenate(
        [ut_agg[users], text_embedding[neg_items]], axis=1)
    compressed_txt_negsams = _mm(text_neg_samples, t_rel_mlp, zeros_n, 2048)

    return (user_preference, items, image_query, text_query, comp_rel_v,
            comp_rel_t, compressed_img_negsams, compressed_txt_negsams,
            v_rel_mlp, t_rel_mlp, image_embedding, text_embedding)


# speedup vs baseline: 1.4559x; 1.0077x over previous
import functools

import jax
import jax.numpy as jnp
from jax import lax
from jax.experimental import pallas as pl
from jax.experimental.pallas import tpu as pltpu
from jax.experimental.pallas import tpu_sc as plsc

N_USERS = 20000
N_ITEMS = 30000
N_NODES = N_USERS + N_ITEMS
E = 800000
D = 64
DELTA = 0.8
N_LAYERS = 2

FCH = 4
LW = 32
FPC = FCH // 2
ER = 6400
EPAD = ER * 128
RPT = ER // 16
NBR = 2
NBATCH = RPT // NBR
N_PAD = 50048
NPT = N_PAD // 16


def _mm_body(x_ref, w_ref, b_ref, o_ref, *, normalize):
    y = jnp.dot(x_ref[...], w_ref[...], preferred_element_type=jnp.float32)
    y = y + b_ref[...]
    if normalize:
        n2 = jnp.sum(y * y, axis=1, keepdims=True)
        y = y * jax.lax.rsqrt(jnp.maximum(n2, 1e-24))
    o_ref[...] = y


def _mm(x, w, b, block_rows, normalize=False):
    m, k = x.shape
    n = w.shape[1]
    assert m % block_rows == 0
    return pl.pallas_call(
        functools.partial(_mm_body, normalize=normalize),
        grid=(m // block_rows,),
        in_specs=[
            pl.BlockSpec((block_rows, k), lambda i: (i, 0)),
            pl.BlockSpec((k, n), lambda i: (0, 0)),
            pl.BlockSpec((1, n), lambda i: (0, 0)),
        ],
        out_specs=pl.BlockSpec((block_rows, n), lambda i: (i, 0)),
        out_shape=jax.ShapeDtypeStruct((m, n), jnp.float32),
    )(x, w, b)


def _bcast_lane(vec, l):
    idx = jnp.full((16, 1), l, dtype=jnp.int32)
    dnums = lax.GatherDimensionNumbers(
        offset_dims=(), collapsed_slice_dims=(0,), start_index_map=(0,))
    return lax.gather(vec, idx, dnums, (1,),
                      mode=lax.GatherScatterMode.PROMISE_IN_BOUNDS)


def _prop_body(x2, zer, srcp, dstr, wr, y2, sidx, didx, wv, rows_bf,
               rows32, acc, gsem, ssem, isem, jsem):
    c = lax.axis_index("c")
    s = lax.axis_index("s")
    row_lo = s * RPT
    node_lo = s * NPT
    EB = NBR * 128

    def fchunk_body(r, _):
        f = c * FPC + r
        fbase = f * N_PAD

        pltpu.sync_copy(zer.at[pl.ds(node_lo, NPT)],
                        acc.at[pl.ds(node_lo, NPT)])
        plsc.subcore_barrier()

        def fire_idx_s(t):
            eb = (row_lo + t * NBR) * 128
            pltpu.async_copy(srcp.at[pl.ds(f * EPAD + eb, EB)],
                             sidx.at[t % 3], isem)

        def drain_idx_s(t):
            eb = (row_lo + t * NBR) * 128
            pltpu.make_async_copy(srcp.at[pl.ds(f * EPAD + eb, EB)],
                                  sidx.at[t % 3], isem).wait()

        def fire_idx_dw(t):
            eb = (row_lo + t * NBR) * 128
            pltpu.async_copy(dstr.at[pl.ds(eb, EB)], didx.at[t % 3], jsem)
            pltpu.async_copy(wr.at[pl.ds(eb, EB)], wv.at[t % 2], jsem)

        def drain_idx_dw(t):
            eb = (row_lo + t * NBR) * 128
            pltpu.make_async_copy(dstr.at[pl.ds(eb, EB)], didx.at[t % 3],
                                  jsem).wait()
            pltpu.make_async_copy(wr.at[pl.ds(eb, EB)], wv.at[t % 2],
                                  jsem).wait()

        def fire_gathers(t):
            for q in range(NBR):
                pltpu.async_copy(x2.at[sidx.at[t % 3, pl.ds(q * 128, 128)]],
                                 rows_bf.at[t % 3, pl.ds(q * 128, 128)], gsem)

        def drain_gathers(t):
            for q in range(NBR):
                pltpu.make_async_copy(
                    x2.at[sidx.at[t % 3, pl.ds(q * 128, 128)]],
                    rows_bf.at[t % 3, pl.ds(q * 128, 128)], gsem).wait()

        def fire_scatter(t):
            for q in range(NBR):
                pltpu.async_copy(rows32.at[t % 2, pl.ds(q * 128, 128)],
                                 acc.at[didx.at[t % 3, pl.ds(q * 128, 128)]],
                                 ssem, add=True)

        def drain_scatter(t):
            for q in range(NBR):
                pltpu.make_async_copy(
                    rows32.at[t % 2, pl.ds(q * 128, 128)],
                    acc.at[didx.at[t % 3, pl.ds(q * 128, 128)]], ssem).wait()

        def compute(t):
            ib3, ib2 = t % 3, t % 2

            @plsc.parallel_loop(0, EB // 16, unroll=2)
            def k_body(k):
                wk = wv[t % 2, pl.ds(k * 16, 16)]
                for l in range(16):
                    i = k * 16 + l
                    wl = _bcast_lane(wk, l)
                    a, b2 = plsc.unpack(rows_bf[ib3, i],
                                        format=plsc.PackFormat.INTERLEAVED)
                    rows32[ib2, i, pl.ds(0, 16)] = a * wl
                    rows32[ib2, i, pl.ds(16, 16)] = b2 * wl

        fire_idx_s(0)
        fire_idx_s(1)
        fire_idx_s(2)
        drain_idx_s(0)
        fire_gathers(0)
        drain_idx_s(1)
        fire_gathers(1)
        fire_idx_dw(0)

        def batch_body(b, _):
            drain_gathers(b)

            @pl.when(b >= 2)
            def _():
                drain_scatter(b - 2)

            @pl.when(b + 3 < NBATCH)
            def _():
                fire_idx_s(b + 3)

            @pl.when(b + 1 < NBATCH)
            def _():
                fire_idx_dw(b + 1)

            @pl.when(b + 2 < NBATCH)
            def _():
                drain_idx_s(b + 2)
                fire_gathers(b + 2)
            drain_idx_dw(b)
            compute(b)
            fire_scatter(b)
            return 0
        lax.fori_loop(0, NBATCH, batch_body, 0)
        drain_scatter(NBATCH - 2)
        drain_scatter(NBATCH - 1)
        plsc.subcore_barrier()

        pltpu.sync_copy(acc.at[pl.ds(node_lo, NPT)],
                        y2.at[pl.ds(fbase + node_lo, NPT)])
        plsc.subcore_barrier()
        return 0

    lax.fori_loop(0, FPC, fchunk_body, 0)


def _propagate_layer(x2b, zer, srcp, dstr, wr):
    return pl.kernel(
        _prop_body,
        out_type=jax.ShapeDtypeStruct((FCH * N_PAD, LW), jnp.float32),
        mesh=plsc.VectorSubcoreMesh(core_axis_name="c", subcore_axis_name="s"),
        compiler_params=pltpu.CompilerParams(use_tc_tiling_on_sc=False,
                                             needs_layout_passes=False),
        scratch_types=[
            pltpu.VMEM((3, NBR * 128), jnp.int32),
            pltpu.VMEM((3, NBR * 128), jnp.int32),
            pltpu.VMEM((2, NBR * 128), jnp.float32),
            pltpu.VMEM((3, NBR * 128, LW), jnp.bfloat16),
            pltpu.VMEM((2, NBR * 128, LW), jnp.float32),
            pltpu.VMEM_SHARED((N_PAD, LW), jnp.float32),
            pltpu.SemaphoreType.DMA,
            pltpu.SemaphoreType.DMA,
            pltpu.SemaphoreType.DMA,
            pltpu.SemaphoreType.DMA,
        ],
    )(x2b, zer, srcp, dstr, wr)


def kernel(edge_index, edge_weight, users, neg_items, image_preference,
           text_preference, image_query, text_query, image_embedding,
           text_embedding, W_img, b_img, W_txt, b_txt, v_rel_mlp, t_rel_mlp,
           image_rel, text_rel, uv_agg, ut_agg):
    b_img2 = b_img.reshape(1, -1)
    b_txt2 = b_txt.reshape(1, -1)
    zeros_n = jnp.zeros((1, D), jnp.float32)

    image_emb = _mm(image_embedding, W_img, b_img2, 2000, normalize=True)
    text_emb = _mm(text_embedding, W_txt, b_txt2, 2000, normalize=True)

    x = jnp.concatenate(
        [jnp.concatenate([image_preference, image_emb], axis=0),
         jnp.concatenate([text_preference, text_emb], axis=0)], axis=1)
    x = jnp.pad(x, ((0, N_PAD - N_NODES), (0, 0)))
    xn = x.reshape(N_PAD, FCH, LW).transpose(1, 0, 2).reshape(
        FCH * N_PAD, LW)
    zer = jnp.zeros((N_PAD, LW), jnp.float32)

    src = edge_index[0]
    dst = edge_index[1]
    pad = EPAD - E
    src_p = jnp.concatenate([src, jnp.zeros((pad,), jnp.int32)])
    dst_p = jnp.concatenate([dst, jnp.zeros((pad,), jnp.int32)])
    w_p = jnp.concatenate([edge_weight[:, 0], jnp.zeros((pad,), jnp.float32)])
    srcp = (src_p.reshape(1, EPAD)
            + (jnp.arange(FCH, dtype=jnp.int32) * N_PAD).reshape(FCH, 1)
            ).reshape(FCH * EPAD)
    dstr = dst_p
    wr = w_p

    for _ in range(N_LAYERS):
        x2b = xn.reshape(-1, 2, 16).transpose(0, 2, 1).reshape(
            -1, LW).astype(jnp.bfloat16)
        y2 = _propagate_layer(x2b, zer, srcp, dstr, wr)
        xn = y2 + DELTA * xn

    x = xn.reshape(FCH, N_PAD, LW).transpose(1, 0, 2).reshape(N_PAD, 128)
    user_preference = x[:N_USERS]
    items = x[N_USERS:N_NODES]

    comp_rel_v = _mm(image_rel, v_rel_mlp, zeros_n, 2000)
    comp_rel_t = _mm(text_rel, t_rel_mlp, zeros_n, 2000)

    image_neg_samples = jnp.concatenate(
        [uv_agg[users], image_embedding[neg_items]], axis=1)
    compressed_img_negsams = _mm(image_neg_samples, v_rel_mlp, zeros_n, 2048)
    text_neg_samples = jnp.concatenate(
        [ut_agg[users], text_embedding[neg_items]], axis=1)
    compressed_txt_negsams = _mm(text_neg_samples, t_rel_mlp, zeros_n, 2048)

    return (user_preference, items, image_query, text_query, comp_rel_v,
            comp_rel_t, compressed_img_negsams, compressed_txt_negsams,
            v_rel_mlp, t_rel_mlp, image_embedding, text_embedding)
